# baseline (speedup 1.0000x reference)
import functools

import jax
import jax.numpy as jnp
from jax import lax
from jax.experimental import pallas as pl
from jax.experimental.pallas import tpu as pltpu
from jax.experimental.pallas import tpu_sc as plsc

_B, _S, _D = 4, 4096, 1024
_NC, _NS = 2, 16
_NW = _NC * _NS
_ROWS_PER_W = _S // _NW
_SUB_ROWS = 8
_N_SUB = _ROWS_PER_W // _SUB_ROWS
_CHUNK = _SUB_ROWS * _D
_NTASK = _N_SUB * _B
_NIN = 4
_NOUT = 2


def _task(k):
    t, b = divmod(k, _B)
    return t, b


def _sc_body(x_hbm, p_hbm, o_hbm,
             xv0, xv1, xv2, xv3, ov0, ov1, pv,
             si0, si1, si2, si3, so0, so1):
    wid = lax.axis_index("s") * _NC + lax.axis_index("c")
    row0 = wid * _ROWS_PER_W
    xbufs = (xv0, xv1, xv2, xv3)
    sins = (si0, si1, si2, si3)
    obufs = (ov0, ov1)
    souts = (so0, so1)
    cin = [None] * _NIN
    cout = [None] * _NOUT

    def in_off(k):
        t, b = _task(k)
        return b, (row0 + t * _SUB_ROWS) * _D

    for k in range(_NIN - 1):
        b, off = in_off(k)
        cin[k % _NIN] = pltpu.async_copy(
            x_hbm.at[b, pl.ds(off, _CHUNK)], xbufs[k % _NIN], sins[k % _NIN])

    for k in range(_NTASK):
        t, b = _task(k)
        off = (row0 + t * _SUB_ROWS) * _D
        if b == 0:
            pltpu.sync_copy(p_hbm.at[pl.ds(off, _CHUNK)], pv)
        nk = k + _NIN - 1
        if nk < _NTASK:
            nb, noff = in_off(nk)
            cin[nk % _NIN] = pltpu.async_copy(
                x_hbm.at[nb, pl.ds(noff, _CHUNK)], xbufs[nk % _NIN], sins[nk % _NIN])
        cb = k % _NIN
        cin[cb].wait()
        xbuf = xbufs[cb]
        ob = k % _NOUT
        if cout[ob] is not None:
            cout[ob].wait()
        obuf = obufs[ob]

        @plsc.parallel_loop(0, _CHUNK, step=16, unroll=8)
        def add_body(i):
            sl = pl.ds(i, 16)
            obuf[sl] = xbuf[sl] + pv[sl]

        cout[ob] = pltpu.async_copy(obuf, o_hbm.at[b, pl.ds(off, _CHUNK)], souts[ob])
    for c in cout:
        if c is not None:
            c.wait()


def kernel(inputs, position_embeddings):
    B, S, D = inputs.shape
    x = inputs.reshape(B, S * D)
    p = position_embeddings.reshape(-1)
    mesh = plsc.VectorSubcoreMesh(core_axis_name="c", subcore_axis_name="s")
    out = pl.kernel(
        _sc_body,
        out_type=jax.ShapeDtypeStruct((B, S * D), jnp.float32),
        mesh=mesh,
        scratch_types=(
            [pltpu.VMEM((_CHUNK,), jnp.float32)] * (_NIN + _NOUT + 1)
            + [pltpu.SemaphoreType.DMA] * (_NIN + _NOUT)
        ),
    )(x, p)
    return out.reshape(B, S, D)


# speedup vs baseline: 4.6924x; 4.6924x over previous
import jax
import jax.numpy as jnp
from jax.experimental import pallas as pl


def _add_kernel(x_ref, p_ref, o_ref):
    o_ref[...] = x_ref[...] + p_ref[...][None, :, :]


def kernel(inputs, position_embeddings):
    B, S, D = inputs.shape
    S_BLK = 2048
    grid = (S // S_BLK, B)
    return pl.pallas_call(
        _add_kernel,
        grid=grid,
        in_specs=[
            pl.BlockSpec((1, S_BLK, D), lambda i, b: (b, i, 0)),
            pl.BlockSpec((S_BLK, D), lambda i, b: (i, 0)),
        ],
        out_specs=pl.BlockSpec((1, S_BLK, D), lambda i, b: (b, i, 0)),
        out_shape=jax.ShapeDtypeStruct((B, S, D), inputs.dtype),
    )(inputs, position_embeddings)
